# baseline (speedup 1.0000x reference)
import jax
import jax.numpy as jnp
from jax import lax
from jax.experimental import pallas as pl
from jax.experimental.pallas import tpu as pltpu
from jax.experimental.pallas import tpu_sc as plsc

DIM = 64
SCALE = 8.0
LANES = 16

_info = plsc.get_sparse_core_info()
NC, NS = _info.num_cores, _info.num_subcores
NW = NC * NS

LC = 4


def _emb_body(table_hbm, idxT_hbm, out_hbm, idx_v, rows_v, outT_v, sem):
    L, B = idxT_hbm.shape
    bpw = B // NW
    n_chunks = L // LC
    wid = lax.axis_index("s") * NC + lax.axis_index("c")
    b0 = wid * bpw

    pltpu.sync_copy(idxT_hbm.at[:, pl.ds(b0, bpw)], idx_v)

    iota = lax.iota(jnp.int32, LANES)

    def chunk_body(ci, carry):
        l0 = ci * LC
        copies = [
            pltpu.async_copy(
                table_hbm.at[idx_v.at[l0 + j]],
                rows_v.at[pl.ds(j * bpw, bpw)],
                sem,
            )
            for j in range(LC)
        ]
        for c in copies:
            c.wait()

        def d_body(d, acc):
            dfull = jnp.full((LANES,), 0, jnp.int32) + d
            for l in range(LC):
                for g in range(bpw // LANES):
                    rowsel = iota + (l * bpw + g * LANES)
                    v = plsc.load_gather(rows_v, [rowsel, dfull])
                    outT_v[l, d, pl.ds(g * LANES, LANES)] = v * SCALE
            return acc

        lax.fori_loop(0, DIM, d_body, 0)

        pltpu.sync_copy(outT_v, out_hbm.at[pl.ds(l0, LC), :, pl.ds(b0, bpw)])
        return carry

    lax.fori_loop(0, n_chunks, chunk_body, 0)


def kernel(x, table):
    B, L = x.shape
    bpw = B // NW
    mesh = plsc.VectorSubcoreMesh(core_axis_name="c", subcore_axis_name="s")
    run = pl.kernel(
        _emb_body,
        mesh=mesh,
        compiler_params=pltpu.CompilerParams(
            use_tc_tiling_on_sc=False, needs_layout_passes=False
        ),
        out_type=jax.ShapeDtypeStruct((L, DIM, B), jnp.float32),
        scratch_types=[
            pltpu.VMEM((L, bpw), jnp.int32),
            pltpu.VMEM((LC * bpw, DIM), jnp.float32),
            pltpu.VMEM((LC, DIM, bpw), jnp.float32),
            pltpu.SemaphoreType.DMA,
        ],
    )
    outT = run(table, x.T)
    return jnp.transpose(outT, (2, 0, 1))


# speedup vs baseline: 1.5861x; 1.5861x over previous
import jax
import jax.numpy as jnp
from jax import lax
from jax.experimental import pallas as pl
from jax.experimental.pallas import tpu as pltpu
from jax.experimental.pallas import tpu_sc as plsc

DIM = 64
SCALE = 8.0
LANES = 16
PITCH = 129

_info = plsc.get_sparse_core_info()
NC, NS = _info.num_cores, _info.num_subcores
NW = NC * NS

LC = 4


def _emb_body(table_hbm, idxT_hbm, out_hbm, idx_v, rows_v, outT_v, sem):
    L, B = idxT_hbm.shape
    bpw = B // NW
    n_chunks = L // LC
    wid = lax.axis_index("s") * NC + lax.axis_index("c")
    b0 = wid * bpw

    pltpu.sync_copy(idxT_hbm.at[:, pl.ds(b0, bpw)], idx_v)

    iota = lax.iota(jnp.int32, LANES)
    dvecs = [iota + d0 for d0 in range(0, DIM, LANES)]
    lvecs = [jnp.full((LANES,), 0, jnp.int32) + l for l in range(LC)]

    def chunk_body(ci, carry):
        l0 = ci * LC
        copies = [
            pltpu.async_copy(
                table_hbm.at[idx_v.at[l0 + j]],
                rows_v.at[pl.ds(j * bpw, bpw)],
                sem,
            )
            for j in range(LC)
        ]
        for c in copies:
            c.wait()

        def b_body(b, acc):
            bfull = jnp.full((LANES,), 0, jnp.int32) + b
            for l in range(LC):
                for k in range(DIM // LANES):
                    v = rows_v[l * bpw + b, pl.ds(k * LANES, LANES)]
                    plsc.store_scatter(
                        outT_v, [lvecs[l], dvecs[k], bfull], v * SCALE
                    )
            return acc

        lax.fori_loop(0, bpw, b_body, 0)

        pltpu.sync_copy(
            outT_v.at[:, :, pl.ds(0, bpw)],
            out_hbm.at[pl.ds(l0, LC), :, pl.ds(b0, bpw)],
        )
        return carry

    lax.fori_loop(0, n_chunks, chunk_body, 0)


def kernel(x, table):
    B, L = x.shape
    bpw = B // NW
    mesh = plsc.VectorSubcoreMesh(core_axis_name="c", subcore_axis_name="s")
    run = pl.kernel(
        _emb_body,
        mesh=mesh,
        compiler_params=pltpu.CompilerParams(
            use_tc_tiling_on_sc=False, needs_layout_passes=False
        ),
        out_type=jax.ShapeDtypeStruct((L, DIM, B), jnp.float32),
        scratch_types=[
            pltpu.VMEM((L, bpw), jnp.int32),
            pltpu.VMEM((LC * bpw, DIM), jnp.float32),
            pltpu.VMEM((LC, DIM, PITCH), jnp.float32),
            pltpu.SemaphoreType.DMA,
        ],
    )
    outT = run(table, x.T)
    return jnp.transpose(outT, (2, 0, 1))
